# baseline (speedup 1.0000x reference)
import functools

import jax
import jax.numpy as jnp
from jax import lax
from jax.experimental import pallas as pl
from jax.experimental.pallas import tpu as pltpu
from jax.experimental.pallas import tpu_sc as plsc

VOCAB = 100000
DIM = 128
B = 4096
L = 50
N = B * L


@functools.lru_cache(maxsize=None)
def _build_gather(nbuf=4, chunk=200):
    info = plsc.get_sparse_core_info()
    nc, ns = info.num_cores, info.num_subcores
    nw = nc * ns
    b_per_w = N // nw
    nchunk = b_per_w // chunk
    assert N % nw == 0 and b_per_w % 8 == 0
    assert nchunk * chunk == b_per_w and nchunk % nbuf == 0 and nchunk >= nbuf

    mesh = plsc.VectorSubcoreMesh(core_axis_name="c", subcore_axis_name="s")

    @functools.partial(
        pl.kernel,
        out_type=jax.ShapeDtypeStruct((N, DIM), jnp.float32),
        mesh=mesh,
        scratch_types=[
            pltpu.VMEM((b_per_w,), jnp.int32),
            pltpu.VMEM((nbuf, chunk, DIM), jnp.float32),
        ]
        + [pltpu.SemaphoreType.DMA] * (2 * nbuf),
    )
    def gather_kernel(table_hbm, idx_hbm, out_hbm, idx_v, rows_v, *sems):
        gsem, osem = sems[:nbuf], sems[nbuf:]
        wid = lax.axis_index("s") * nc + lax.axis_index("c")
        base = wid * b_per_w
        pltpu.sync_copy(idx_hbm.at[pl.ds(base, b_per_w)], idx_v)

        def start_gather(i, b):
            pltpu.async_copy(
                table_hbm.at[idx_v.at[pl.ds(i * chunk, chunk)]],
                rows_v.at[b],
                gsem[b],
            )

        look = nbuf // 2
        for c in range(look):
            start_gather(c, c)

        @pl.loop(0, nchunk, step=nbuf)
        def _(g):
            for b in range(nbuf):
                i = g + b
                pltpu.make_async_copy(
                    table_hbm.at[idx_v.at[pl.ds(0, chunk)]], rows_v.at[b], gsem[b]
                ).wait()
                pltpu.async_copy(
                    rows_v.at[b], out_hbm.at[pl.ds(base + i * chunk, chunk)], osem[b]
                )
                j = i + look
                bj = (b + look) % nbuf

                @pl.when(jnp.logical_and(j >= nbuf, j < nchunk))
                def _():
                    pltpu.make_async_copy(
                        rows_v.at[bj],
                        out_hbm.at[pl.ds(base, chunk)],
                        osem[bj],
                    ).wait()

                @pl.when(j < nchunk)
                def _():
                    start_gather(j, bj)

        for b in range(nbuf):
            pltpu.make_async_copy(
                rows_v.at[b], out_hbm.at[pl.ds(base, chunk)], osem[b]
            ).wait()

    return gather_kernel


def kernel(state, input_token, table):
    idx = input_token.reshape(N).astype(jnp.int32)
    rows = _build_gather()(table, idx)
    return (state, rows.reshape(B, L, DIM))


# speedup vs baseline: 1.7856x; 1.7856x over previous
import functools

import jax
import jax.numpy as jnp
from jax import lax
from jax.experimental import pallas as pl
from jax.experimental.pallas import tpu as pltpu
from jax.experimental.pallas import tpu_sc as plsc

VOCAB = 100000
DIM = 128
B = 4096
L = 50
LP = 56
N = B * L


@functools.lru_cache(maxsize=None)
def _build_gather(nbuf=8):
    info = plsc.get_sparse_core_info()
    nc, ns = info.num_cores, info.num_subcores
    nw = nc * ns
    s_per_w = B // nw
    assert B % nw == 0 and s_per_w % nbuf == 0

    mesh = plsc.VectorSubcoreMesh(core_axis_name="c", subcore_axis_name="s")

    @functools.partial(
        pl.kernel,
        out_type=jax.ShapeDtypeStruct((B, L, DIM), jnp.float32),
        mesh=mesh,
        scratch_types=[
            pltpu.VMEM((s_per_w, L), jnp.int32),
            pltpu.VMEM((nbuf, L, DIM), jnp.float32),
        ]
        + [pltpu.SemaphoreType.DMA] * (2 * nbuf),
    )
    def gather_kernel(table_hbm, idx_hbm, out_hbm, idx_v, rows_v, *sems):
        gsem, osem = sems[:nbuf], sems[nbuf:]
        wid = lax.axis_index("s") * nc + lax.axis_index("c")
        sbase = wid * s_per_w
        pltpu.sync_copy(idx_hbm.at[pl.ds(sbase, s_per_w)], idx_v)

        def start_gather(i, b):
            pltpu.async_copy(
                table_hbm.at[idx_v.at[i]], rows_v.at[b], gsem[b]
            )

        look = nbuf // 2
        for c in range(look):
            start_gather(c, c)

        @pl.loop(0, s_per_w, step=nbuf)
        def _(g):
            for b in range(nbuf):
                i = g + b
                pltpu.make_async_copy(
                    table_hbm.at[idx_v.at[0]], rows_v.at[b], gsem[b]
                ).wait()
                pltpu.async_copy(
                    rows_v.at[b],
                    out_hbm.at[sbase + i],
                    osem[b],
                )
                j = i + look
                bj = (b + look) % nbuf

                @pl.when(jnp.logical_and(j >= nbuf, j < s_per_w))
                def _():
                    pltpu.make_async_copy(
                        rows_v.at[bj],
                        out_hbm.at[sbase],
                        osem[bj],
                    ).wait()

                @pl.when(j < s_per_w)
                def _():
                    start_gather(j, bj)

        for b in range(nbuf):
            pltpu.make_async_copy(
                rows_v.at[b], out_hbm.at[sbase], osem[b]
            ).wait()

    return gather_kernel


def kernel(state, input_token, table):
    hidden = _build_gather()(table, input_token.astype(jnp.int32))
    return (state, hidden)


# speedup vs baseline: 3.0509x; 1.7086x over previous
import functools

import jax
import jax.numpy as jnp
from jax import lax
from jax.experimental import pallas as pl
from jax.experimental.pallas import tpu as pltpu
from jax.experimental.pallas import tpu_sc as plsc

VOCAB = 100000
DIM = 128
B = 4096
L = 50
N = B * L


@functools.lru_cache(maxsize=None)
def _build_gather(nbuf=4, chunk=200):
    info = plsc.get_sparse_core_info()
    nc, ns = info.num_cores, info.num_subcores
    nw = nc * ns
    b_per_w = N // nw
    nchunk = b_per_w // chunk
    assert N % nw == 0 and b_per_w % 8 == 0 and chunk % 8 == 0
    assert nchunk * chunk == b_per_w and nchunk % nbuf == 0 and nchunk >= nbuf

    mesh = plsc.VectorSubcoreMesh(core_axis_name="c", subcore_axis_name="s")

    @functools.partial(
        pl.kernel,
        out_type=jax.ShapeDtypeStruct((N, DIM), jnp.float32),
        mesh=mesh,
        scratch_types=[
            pltpu.VMEM((b_per_w,), jnp.int32),
            pltpu.VMEM((nbuf, chunk, DIM), jnp.float32),
        ]
        + [pltpu.SemaphoreType.DMA] * (2 * nbuf),
    )
    def gather_kernel(table_hbm, idx_hbm, out_hbm, idx_v, rows_v, *sems):
        gsem, osem = sems[:nbuf], sems[nbuf:]
        wid = lax.axis_index("s") * nc + lax.axis_index("c")
        base = wid * b_per_w
        pltpu.sync_copy(idx_hbm.at[pl.ds(base, b_per_w)], idx_v)

        def start_gather(i, b):
            pltpu.async_copy(
                table_hbm.at[idx_v.at[pl.ds(i * chunk, chunk)]],
                rows_v.at[b],
                gsem[b],
            )

        look = nbuf // 2
        for c in range(look):
            start_gather(c, c)

        @pl.loop(0, nchunk, step=nbuf)
        def _(g):
            for b in range(nbuf):
                i = g + b
                pltpu.make_async_copy(
                    table_hbm.at[idx_v.at[pl.ds(0, chunk)]], rows_v.at[b], gsem[b]
                ).wait()
                pltpu.async_copy(
                    rows_v.at[b], out_hbm.at[pl.ds(base + i * chunk, chunk)], osem[b]
                )
                j = i + look
                bj = (b + look) % nbuf

                @pl.when(jnp.logical_and(j >= nbuf, j < nchunk))
                def _():
                    pltpu.make_async_copy(
                        rows_v.at[bj],
                        out_hbm.at[pl.ds(base, chunk)],
                        osem[bj],
                    ).wait()

                @pl.when(j < nchunk)
                def _():
                    start_gather(j, bj)

        for b in range(nbuf):
            pltpu.make_async_copy(
                rows_v.at[b], out_hbm.at[pl.ds(base, chunk)], osem[b]
            ).wait()

    return gather_kernel


def kernel(state, input_token, table):
    idx_t = input_token.astype(jnp.int32).T.reshape(N)
    rows = _build_gather()(table, idx_t)
    hidden = rows.reshape(L, B, DIM).transpose(1, 0, 2)
    return (state, hidden)


# speedup vs baseline: 3.0654x; 1.0048x over previous
import functools

import jax
import jax.numpy as jnp
from jax import lax
from jax.experimental import pallas as pl
from jax.experimental.pallas import tpu as pltpu
from jax.experimental.pallas import tpu_sc as plsc

VOCAB = 100000
DIM = 128
B = 4096
L = 50
N = B * L


@functools.lru_cache(maxsize=None)
def _build_gather(nbuf=8, chunk=80):
    info = plsc.get_sparse_core_info()
    nc, ns = info.num_cores, info.num_subcores
    nw = nc * ns
    b_per_w = N // nw
    nchunk = b_per_w // chunk
    assert N % nw == 0 and b_per_w % 8 == 0 and chunk % 8 == 0
    assert nchunk * chunk == b_per_w and nchunk % nbuf == 0 and nchunk >= nbuf

    mesh = plsc.VectorSubcoreMesh(core_axis_name="c", subcore_axis_name="s")

    @functools.partial(
        pl.kernel,
        out_type=jax.ShapeDtypeStruct((N, DIM), jnp.float32),
        mesh=mesh,
        scratch_types=[
            pltpu.VMEM((b_per_w,), jnp.int32),
            pltpu.VMEM((nbuf, chunk, DIM), jnp.float32),
        ]
        + [pltpu.SemaphoreType.DMA] * (2 * nbuf),
    )
    def gather_kernel(table_hbm, idx_hbm, out_hbm, idx_v, rows_v, *sems):
        gsem, osem = sems[:nbuf], sems[nbuf:]
        wid = lax.axis_index("s") * nc + lax.axis_index("c")
        base = wid * b_per_w
        pltpu.sync_copy(idx_hbm.at[pl.ds(base, b_per_w)], idx_v)

        def start_gather(i, b):
            pltpu.async_copy(
                table_hbm.at[idx_v.at[pl.ds(i * chunk, chunk)]],
                rows_v.at[b],
                gsem[b],
            )

        look = nbuf // 2
        for c in range(look):
            start_gather(c, c)

        @pl.loop(0, nchunk, step=nbuf)
        def _(g):
            for b in range(nbuf):
                i = g + b
                pltpu.make_async_copy(
                    table_hbm.at[idx_v.at[pl.ds(0, chunk)]], rows_v.at[b], gsem[b]
                ).wait()
                pltpu.async_copy(
                    rows_v.at[b], out_hbm.at[pl.ds(base + i * chunk, chunk)], osem[b]
                )
                j = i + look
                bj = (b + look) % nbuf

                @pl.when(jnp.logical_and(j >= nbuf, j < nchunk))
                def _():
                    pltpu.make_async_copy(
                        rows_v.at[bj],
                        out_hbm.at[pl.ds(base, chunk)],
                        osem[bj],
                    ).wait()

                @pl.when(j < nchunk)
                def _():
                    start_gather(j, bj)

        for b in range(nbuf):
            pltpu.make_async_copy(
                rows_v.at[b], out_hbm.at[pl.ds(base, chunk)], osem[b]
            ).wait()

    return gather_kernel


def kernel(state, input_token, table):
    idx_t = input_token.astype(jnp.int32).T.reshape(N)
    rows = _build_gather()(table, idx_t)
    hidden = rows.reshape(L, B, DIM).transpose(1, 0, 2)
    return (state, hidden)
